# baseline (speedup 1.0000x reference)
import jax
import jax.numpy as jnp
from jax.experimental import pallas as pl
from jax.experimental.pallas import tpu as pltpu

B, N = 16, 1024
RB = 512


def _body(cmask_ref, rmask_ref, true_ref, pred_ref, sum_ref, cnt_ref):
    i = pl.program_id(0)
    j = pl.program_id(1)

    x = pred_ref[0]
    z = (true_ref[0] == 1).astype(jnp.float32)
    rm = rmask_ref[0, 0]
    cm = cmask_ref[0, 0]
    m2 = rm[:, None] * cm[None, :]

    pe = jnp.maximum(x, 0.0) - x * z + jnp.log1p(jnp.exp(-jnp.abs(x)))
    s = jnp.sum(pe * m2)
    c = jnp.sum(m2)

    @pl.when(jnp.logical_and(i == 0, j == 0))
    def _():
        sum_ref[0, 0] = 0.0
        cnt_ref[0, 0] = 0.0

    sum_ref[0, 0] += s
    cnt_ref[0, 0] += c


def kernel(mask, edge_features_true, edge_features_pred):
    maskf = mask.astype(jnp.float32).reshape(B, 1, N)
    t = edge_features_true.astype(jnp.int32)
    x = edge_features_pred

    grid = (B, N // RB)
    out = pl.pallas_call(
        _body,
        grid=grid,
        in_specs=[
            pl.BlockSpec((1, 1, N), lambda i, j: (i, 0, 0)),
            pl.BlockSpec((1, 1, RB), lambda i, j: (i, 0, j)),
            pl.BlockSpec((1, RB, N), lambda i, j: (i, j, 0)),
            pl.BlockSpec((1, RB, N), lambda i, j: (i, j, 0)),
        ],
        out_specs=[
            pl.BlockSpec((1, 1), lambda i, j: (0, 0), memory_space=pltpu.SMEM),
            pl.BlockSpec((1, 1), lambda i, j: (0, 0), memory_space=pltpu.SMEM),
        ],
        out_shape=[
            jax.ShapeDtypeStruct((1, 1), jnp.float32),
            jax.ShapeDtypeStruct((1, 1), jnp.float32),
        ],
    )(maskf, maskf, t, x)
    s, c = out
    return s[0, 0] / jnp.maximum(c[0, 0], 1.0)
